# Initial kernel scaffold; baseline (speedup 1.0000x reference)
import math

import jax
import jax.numpy as jnp
from jax import lax
from jax.experimental import pallas as pl
from jax.experimental.pallas import tpu as pltpu
from jax.experimental.pallas import tpu_sc as plsc

LQ = 2048
NH = 12
CH = 64
TOPK = 32

LANES = 16
NC, NS = 2, 16
NW = NC * NS
BLK = LANES
NB = LQ // BLK
NTASK = NH * NB
TASKS_PER_W = NTASK // NW
ROWS = BLK * TOPK
IDX_CHUNKS = ROWS // 128


def _sc_attention(q_hbm, k_hbm, v_hbm, idx_hbm, out_hbm,
                  idxv, qbuf, kbuf, vbuf, obuf, sem):
    cid = lax.axis_index("c")
    sid = lax.axis_index("s")
    wid = sid * NC + cid

    lanes = lax.iota(jnp.int32, LANES)
    rowbase = lanes * TOPK
    scale = 1.0 / math.sqrt(CH)

    def task_body(i, carry):
        task = wid * TASKS_PER_W + i
        h = task // NB
        lb = task % NB
        l0 = lb * BLK

        pltpu.sync_copy(idx_hbm.at[h, lb], idxv)
        pltpu.sync_copy(q_hbm.at[h, pl.ds(l0, BLK)], qbuf)
        copies = []
        for j in range(IDX_CHUNKS):
            copies.append(pltpu.async_copy(
                k_hbm.at[idxv.at[j]], kbuf.at[pl.ds(j * 128, 128)], sem))
        for j in range(IDX_CHUNKS):
            copies.append(pltpu.async_copy(
                v_hbm.at[idxv.at[j]], vbuf.at[pl.ds(j * 128, 128)], sem))
        for cp in copies:
            cp.wait()

        def score_step(c, S):
            ccol = lax.broadcast(c, (LANES,))
            qcol = plsc.load_gather(qbuf, [lanes, ccol])
            return tuple(
                S[t] + qcol * plsc.load_gather(kbuf, [rowbase + t, ccol])
                for t in range(TOPK))

        S0 = tuple(jnp.zeros((LANES,), jnp.float32) for _ in range(TOPK))
        S = lax.fori_loop(0, CH, score_step, S0)
        S = [s * scale for s in S]

        m = S[0]
        for t in range(1, TOPK):
            m = jnp.maximum(m, S[t])
        e = [jnp.exp(s - m) for s in S]
        den = e[0]
        for t in range(1, TOPK):
            den = den + e[t]
        inv = 1.0 / den
        w = tuple(et * inv for et in e)

        def out_step(c, carry2):
            ccol = lax.broadcast(c, (LANES,))
            acc = w[0] * plsc.load_gather(vbuf, [rowbase, ccol])
            for t in range(1, TOPK):
                acc = acc + w[t] * plsc.load_gather(vbuf, [rowbase + t, ccol])
            plsc.store_scatter(obuf, [lanes, ccol], acc)
            return carry2

        lax.fori_loop(0, CH, out_step, 0)
        pltpu.sync_copy(obuf, out_hbm.at[h, pl.ds(l0, BLK)])
        return carry

    lax.fori_loop(0, TASKS_PER_W, task_body, 0)


@jax.jit
def kernel(query, key, value, pos):
    q_t = query[0].transpose(1, 0, 2)
    k2d = key[0].reshape(LQ * NH, CH)
    v2d = value[0].reshape(LQ * NH, CH)
    rowidx = pos[0] * NH + jnp.arange(NH, dtype=jnp.int32)[None, :, None]
    rowidx = rowidx.transpose(1, 0, 2).reshape(NH, NB, IDX_CHUNKS, 128)

    mesh = plsc.VectorSubcoreMesh(core_axis_name="c", subcore_axis_name="s")
    call = pl.kernel(
        _sc_attention,
        out_type=jax.ShapeDtypeStruct((NH, LQ, CH), jnp.float32),
        mesh=mesh,
        scratch_types=[
            pltpu.VMEM((IDX_CHUNKS, 128), jnp.int32),
            pltpu.VMEM((BLK, CH), jnp.float32),
            pltpu.VMEM((ROWS, CH), jnp.float32),
            pltpu.VMEM((ROWS, CH), jnp.float32),
            pltpu.VMEM((BLK, CH), jnp.float32),
            pltpu.SemaphoreType.DMA,
        ],
    )
    out_t = call(q_t, k2d, v2d, rowidx)
    return out_t.transpose(1, 0, 2)[None]


# speedup vs baseline: 11.2073x; 11.2073x over previous
import math

import jax
import jax.numpy as jnp
from jax import lax
from jax.experimental import pallas as pl
from jax.experimental.pallas import tpu as pltpu
from jax.experimental.pallas import tpu_sc as plsc

LQ = 2048
NH = 12
CH = 64
TOPK = 32

LANES = 16
NC, NS = 2, 16
NW = NC * NS
BLK = LANES
NB = LQ // BLK
NTASK = NH * NB
TASKS_PER_W = NTASK // NW
ROWS = BLK * TOPK
IDX_CHUNKS = ROWS // 128


def _sc_attention(q_hbm, kv_hbm, idx_hbm, out_hbm,
                  idxv, qbuf, kvbuf, obuf, sem):
    cid = lax.axis_index("c")
    sid = lax.axis_index("s")
    wid = sid * NC + cid

    lanes = lax.iota(jnp.int32, LANES)
    rowbase = lanes * TOPK
    scale = 1.0 / math.sqrt(CH)

    def task_body(i, carry):
        task = wid * TASKS_PER_W + i
        h = task // NB
        lb = task % NB
        l0 = lb * BLK

        pltpu.sync_copy(idx_hbm.at[h, lb], idxv)
        pltpu.sync_copy(q_hbm.at[h, pl.ds(l0, BLK)], qbuf)
        copies = []
        for j in range(IDX_CHUNKS):
            copies.append(pltpu.async_copy(
                kv_hbm.at[idxv.at[j]], kvbuf.at[pl.ds(j * 128, 128)], sem))
        for cp in copies:
            cp.wait()

        def score_step(c, S):
            ccol = lax.broadcast(c, (LANES,))
            qcol = plsc.load_gather(qbuf, [lanes, ccol])
            return tuple(
                S[t] + qcol * plsc.load_gather(kvbuf, [rowbase + t, ccol])
                for t in range(TOPK))

        S0 = tuple(jnp.zeros((LANES,), jnp.float32) for _ in range(TOPK))
        S = lax.fori_loop(0, CH, score_step, S0)
        S = [s * scale for s in S]

        m = S[0]
        for t in range(1, TOPK):
            m = jnp.maximum(m, S[t])
        e = [jnp.exp(s - m) for s in S]
        den = e[0]
        for t in range(1, TOPK):
            den = den + e[t]
        inv = 1.0 / den
        w = tuple(et * inv for et in e)

        def out_step(c, carry2):
            ccol = lax.broadcast(c, (LANES,))
            vcol = ccol + CH
            acc = w[0] * plsc.load_gather(kvbuf, [rowbase, vcol])
            for t in range(1, TOPK):
                acc = acc + w[t] * plsc.load_gather(kvbuf, [rowbase + t, vcol])
            plsc.store_scatter(obuf, [lanes, ccol], acc)
            return carry2

        lax.fori_loop(0, CH, out_step, 0)
        pltpu.sync_copy(obuf, out_hbm.at[h, pl.ds(l0, BLK)])
        return carry

    lax.fori_loop(0, TASKS_PER_W, task_body, 0)


@jax.jit
def kernel(query, key, value, pos):
    q_t = query[0].transpose(1, 0, 2)
    kv2d = jnp.concatenate(
        [key[0].reshape(LQ * NH, CH), value[0].reshape(LQ * NH, CH)], axis=-1)
    rowidx = pos[0] * NH + jnp.arange(NH, dtype=jnp.int32)[None, :, None]
    rowidx = rowidx.transpose(1, 0, 2).reshape(NH, NB, IDX_CHUNKS, 128)

    mesh = plsc.VectorSubcoreMesh(core_axis_name="c", subcore_axis_name="s")
    call = pl.kernel(
        _sc_attention,
        out_type=jax.ShapeDtypeStruct((NH, LQ, CH), jnp.float32),
        mesh=mesh,
        compiler_params=pltpu.CompilerParams(needs_layout_passes=False),
        scratch_types=[
            pltpu.VMEM((IDX_CHUNKS, 128), jnp.int32),
            pltpu.VMEM((BLK, CH), jnp.float32),
            pltpu.VMEM((ROWS, 2 * CH), jnp.float32),
            pltpu.VMEM((BLK, CH), jnp.float32),
            pltpu.SemaphoreType.DMA,
        ],
    )
    out_t = call(q_t, kv2d, rowidx)
    return out_t.transpose(1, 0, 2)[None]


# speedup vs baseline: 14.7196x; 1.3134x over previous
import math

import jax
import jax.numpy as jnp
from jax import lax
from jax.experimental import pallas as pl
from jax.experimental.pallas import tpu as pltpu
from jax.experimental.pallas import tpu_sc as plsc

LQ = 2048
NH = 12
CH = 64
TOPK = 32

LANES = 16
NC, NS = 2, 16
NW = NC * NS
BLK = LANES
NB = LQ // BLK
NTASK = NH * NB
TASKS_PER_W = NTASK // NW
ROWS = BLK * TOPK
IDX_CHUNKS = ROWS // 128


def _tree_sum(vals):
    vals = list(vals)
    while len(vals) > 1:
        nxt = [vals[i] + vals[i + 1] for i in range(0, len(vals) - 1, 2)]
        if len(vals) % 2:
            nxt.append(vals[-1])
        vals = nxt
    return vals[0]


def _sc_attention(q_hbm, kv_hbm, idx_hbm, out_hbm,
                  idxv, qbuf, kvbuf, sbuf, obuf, sem):
    cid = lax.axis_index("c")
    sid = lax.axis_index("s")
    wid = sid * NC + cid

    lanes = lax.iota(jnp.int32, LANES)
    rowbase = lanes * TOPK
    scale = 1.0 / math.sqrt(CH)

    def task_body(i, carry):
        task = wid * TASKS_PER_W + i
        h = task // NB
        lb = task % NB
        l0 = lb * BLK

        pltpu.sync_copy(idx_hbm.at[h, lb], idxv)
        pltpu.sync_copy(q_hbm.at[h, pl.ds(l0, BLK)], qbuf)
        copies = []
        for j in range(IDX_CHUNKS):
            copies.append(pltpu.async_copy(
                kv_hbm.at[idxv.at[j]], kvbuf.at[pl.ds(j * 128, 128)], sem))
        for cp in copies:
            cp.wait()

        zero = jnp.zeros((LANES,), jnp.float32)
        for t in range(TOPK):
            sbuf[t, :] = zero

        @plsc.parallel_loop(0, CH)
        def score_step(c):
            ccol = lax.broadcast(c, (LANES,))
            qcol = plsc.load_gather(qbuf, [lanes, ccol])
            for t in range(TOPK):
                plsc.addupdate(
                    sbuf.at[t],
                    qcol * plsc.load_gather(kvbuf, [rowbase + t, ccol]))
        S = [sbuf[t, :] * scale for t in range(TOPK)]

        ms = [jnp.maximum(S[2 * t], S[2 * t + 1]) for t in range(TOPK // 2)]
        while len(ms) > 1:
            ms = [jnp.maximum(ms[2 * i], ms[2 * i + 1])
                  for i in range(len(ms) // 2)]
        m = ms[0]
        e = [jnp.exp(s - m) for s in S]
        inv = 1.0 / _tree_sum(e)
        w = tuple(et * inv for et in e)

        @plsc.parallel_loop(0, CH)
        def out_a(c):
            ccol = lax.broadcast(c, (LANES,))
            vcol = ccol + CH
            acc = _tree_sum(
                w[t] * plsc.load_gather(kvbuf, [rowbase + t, vcol])
                for t in range(TOPK // 2))
            plsc.store_scatter(obuf, [lanes, ccol], acc)

        @plsc.parallel_loop(0, CH)
        def out_b(c):
            ccol = lax.broadcast(c, (LANES,))
            vcol = ccol + CH
            acc = _tree_sum(
                w[t] * plsc.load_gather(kvbuf, [rowbase + t, vcol])
                for t in range(TOPK // 2, TOPK))
            plsc.addupdate_scatter(obuf, [lanes, ccol], acc)
        pltpu.sync_copy(obuf, out_hbm.at[h, pl.ds(l0, BLK)])
        return carry

    lax.fori_loop(0, TASKS_PER_W, task_body, 0)


@jax.jit
def kernel(query, key, value, pos):
    q_t = query[0].transpose(1, 0, 2)
    kv2d = jnp.concatenate(
        [key[0].reshape(LQ * NH, CH), value[0].reshape(LQ * NH, CH)], axis=-1)
    rowidx = pos[0] * NH + jnp.arange(NH, dtype=jnp.int32)[None, :, None]
    rowidx = rowidx.transpose(1, 0, 2).reshape(NH, NB, IDX_CHUNKS, 128)

    mesh = plsc.VectorSubcoreMesh(core_axis_name="c", subcore_axis_name="s")
    call = pl.kernel(
        _sc_attention,
        out_type=jax.ShapeDtypeStruct((NH, LQ, CH), jnp.float32),
        mesh=mesh,
        compiler_params=pltpu.CompilerParams(needs_layout_passes=False),
        scratch_types=[
            pltpu.VMEM((IDX_CHUNKS, 128), jnp.int32),
            pltpu.VMEM((BLK, CH), jnp.float32),
            pltpu.VMEM((ROWS, 2 * CH), jnp.float32),
            pltpu.VMEM((TOPK, LANES), jnp.float32),
            pltpu.VMEM((BLK, CH), jnp.float32),
            pltpu.SemaphoreType.DMA,
        ],
    )
    out_t = call(q_t, kv2d, rowidx)
    return out_t.transpose(1, 0, 2)[None]
